# baseline (speedup 1.0000x reference)
import functools

import jax
import jax.numpy as jnp
from jax import lax
from jax.experimental import pallas as pl
from jax.experimental.pallas import tpu as pltpu
from jax.experimental.pallas import tpu_sc as plsc


def _build(B, D, dtype):
    info = plsc.get_sparse_core_info()
    NC, NS = info.num_cores, info.num_subcores
    NW = NC * NS
    assert B % NW == 0
    b_per_w = B // NW
    C = 1280
    assert b_per_w % C == 0
    n_chunks = b_per_w // C
    assert n_chunks % 2 == 0

    mesh = plsc.VectorSubcoreMesh(core_axis_name="c", subcore_axis_name="s")

    @functools.partial(
        pl.kernel,
        mesh=mesh,
        out_type=jax.ShapeDtypeStruct((B, D), dtype),
        scratch_types=[
            pltpu.VMEM((b_per_w,), jnp.int32),
            pltpu.VMEM((C, D), dtype),
            pltpu.VMEM((C, D), dtype),
            pltpu.SemaphoreType.DMA,
            pltpu.SemaphoreType.DMA,
            pltpu.SemaphoreType.DMA,
            pltpu.SemaphoreType.DMA,
        ],
        compiler_params=pltpu.CompilerParams(use_tc_tiling_on_sc=False),
    )
    def emb_kernel(x_hbm, table_hbm, out_hbm, idx_v, rows0, rows1,
                   sg0, sg1, so0, so1):
        wid = lax.axis_index("s") * NC + lax.axis_index("c")
        base = wid * b_per_w
        pltpu.sync_copy(x_hbm.at[pl.ds(base, b_per_w)], idx_v)

        def gather(i, rows, sem):
            return pltpu.make_async_copy(
                table_hbm.at[idx_v.at[pl.ds(i * C, C)]], rows, sem)

        def store(i, rows, sem):
            return pltpu.make_async_copy(
                rows, out_hbm.at[pl.ds(base + i * C, C)], sem)

        gather(0, rows0, sg0).start()

        def step(j, carry):
            for p in range(2):
                i = 2 * j + p
                buf, sgp, sop = (rows0, sg0, so0) if p == 0 else (rows1, sg1, so1)
                nbuf, sgn, son = (rows1, sg1, so1) if p == 0 else (rows0, sg0, so0)
                gather(i, buf, sgp).wait()
                store(i, buf, sop).start()

                @pl.when(i + 1 < n_chunks)
                def _():
                    @pl.when(i >= 1)
                    def _():
                        store(i - 1, nbuf, son).wait()

                    gather(i + 1, nbuf, sgn).start()

            return carry

        lax.fori_loop(0, n_chunks // 2, step, 0)
        store(n_chunks - 2, rows0, so0).wait()
        store(n_chunks - 1, rows1, so1).wait()

    return emb_kernel


def kernel(x, table):
    orig_shape = x.shape
    xf = x.reshape(-1).astype(jnp.int32)
    B = xf.shape[0]
    D = table.shape[1]
    out = _build(B, D, table.dtype)(xf, table)
    return out.reshape(*orig_shape, D)


# speedup vs baseline: 1.3663x; 1.3663x over previous
import functools

import jax
import jax.numpy as jnp
from jax import lax
from jax.experimental import pallas as pl
from jax.experimental.pallas import tpu as pltpu
from jax.experimental.pallas import tpu_sc as plsc


def _build(B, D, V, dtype):
    info = plsc.get_sparse_core_info()
    NC, NS, L = info.num_cores, info.num_subcores, info.num_lanes
    NW = NC * NS
    PACK = 128 // D
    BLK = 128
    assert B % (NW * BLK) == 0
    n_blocks = B // (NW * BLK)
    assert n_blocks % 2 == 0
    b_per_w = n_blocks * BLK
    n_groups = BLK // L

    mesh = plsc.VectorSubcoreMesh(core_axis_name="c", subcore_axis_name="s")

    @functools.partial(
        pl.kernel,
        mesh=mesh,
        out_type=jax.ShapeDtypeStruct((B // 16384, D, 16384), dtype),
        scratch_types=[
            pltpu.VMEM((b_per_w,), jnp.int32),
            pltpu.VMEM((b_per_w,), jnp.int32),
            pltpu.VMEM((BLK, 128), dtype),
            pltpu.VMEM((BLK, 128), dtype),
            pltpu.VMEM((D, BLK), dtype),
            pltpu.VMEM((D, BLK), dtype),
            pltpu.SemaphoreType.DMA,
            pltpu.SemaphoreType.DMA,
            pltpu.SemaphoreType.DMA,
            pltpu.SemaphoreType.DMA,
        ],
        compiler_params=pltpu.CompilerParams(
            use_tc_tiling_on_sc=True, needs_layout_passes=False),
    )
    def emb_kernel(x_hbm, table_hbm, out_hbm, idx_v, idxq_v,
                   rows0, rows1, blk0, blk1, sg0, sg1, so0, so1):
        wid = lax.axis_index("s") * NC + lax.axis_index("c")
        g0 = wid * n_blocks
        pltpu.sync_copy(x_hbm.at[pl.ds(g0 * BLK, b_per_w)], idx_v)

        def quo(k, carry):
            v = idx_v[pl.ds(k * L, L)]
            idxq_v[pl.ds(k * L, L)] = lax.shift_right_logical(v, 2)
            return carry

        lax.fori_loop(0, b_per_w // L, quo, 0)

        jj = lax.iota(jnp.int32, L)

        def gather(i, rows, sem):
            return pltpu.make_async_copy(
                table_hbm.at[idxq_v.at[pl.ds(i * BLK, BLK)]], rows, sem)

        def store(i, blk, sem):
            g = g0 + i
            s = g // (16384 // BLK)
            b0 = (g % (16384 // BLK)) * BLK
            return pltpu.make_async_copy(
                blk, out_hbm.at[s, :, pl.ds(b0, BLK)], sem)

        def extract(i, rows, blk):
            for g8 in range(n_groups):
                v = idx_v[pl.ds(i * BLK + g8 * L, L)]
                col0 = (v & (PACK - 1)) * D
                rowv = jj + g8 * L
                for d in range(D):
                    vals = plsc.load_gather(rows, [rowv, col0 + d])
                    blk[d, pl.ds(g8 * L, L)] = vals

        gather(0, rows0, sg0).start()

        def step(k, carry):
            for p in range(2):
                i = 2 * k + p
                rows, blk, sgp, sop = (
                    (rows0, blk0, sg0, so0) if p == 0 else (rows1, blk1, sg1, so1))
                nrows, sgn, son = (
                    (rows1, sg1, so1) if p == 0 else (rows0, sg0, so0))
                nblk = blk1 if p == 0 else blk0

                @pl.when(i + 1 < n_blocks)
                def _():
                    @pl.when(i >= 1)
                    def _():
                        store(i - 1, nblk, son).wait()

                    gather(i + 1, nrows, sgn).start()

                gather(i, rows, sgp).wait()
                extract(i, rows, blk)
                store(i, blk, sop).start()

            return carry

        lax.fori_loop(0, n_blocks // 2, step, 0)
        store(n_blocks - 2, blk0, so0).wait()
        store(n_blocks - 1, blk1, so1).wait()

    return emb_kernel


def kernel(x, table):
    B0, S = x.shape
    V, D = table.shape
    B = B0 * S
    xf = jnp.transpose(x).reshape(-1).astype(jnp.int32)
    table4 = table.reshape(V * D // 128, 128)
    out_t = _build(B, D, V, table.dtype)(xf, table4)
    return jnp.transpose(out_t, (2, 0, 1))


# speedup vs baseline: 1.3727x; 1.0047x over previous
import functools

import jax
import jax.numpy as jnp
from jax import lax
from jax.experimental import pallas as pl
from jax.experimental.pallas import tpu as pltpu
from jax.experimental.pallas import tpu_sc as plsc


def _build(B, D, V, dtype):
    info = plsc.get_sparse_core_info()
    NC, NS, L = info.num_cores, info.num_subcores, info.num_lanes
    NW = NC * NS
    PACK = 128 // D
    BLK = 128
    assert B % (NW * BLK) == 0
    n_blocks = B // (NW * BLK)
    assert n_blocks % 2 == 0
    b_per_w = n_blocks * BLK
    n_groups = BLK // L

    mesh = plsc.VectorSubcoreMesh(core_axis_name="c", subcore_axis_name="s")

    @functools.partial(
        pl.kernel,
        mesh=mesh,
        out_type=jax.ShapeDtypeStruct((B // 16384, D, 16384), dtype),
        scratch_types=[
            pltpu.VMEM((b_per_w,), jnp.int32),
            pltpu.VMEM((b_per_w,), jnp.int32),
            pltpu.VMEM((BLK, 128), dtype),
            pltpu.VMEM((BLK, 128), dtype),
            pltpu.VMEM((D, BLK), dtype),
            pltpu.VMEM((D, BLK), dtype),
            pltpu.SemaphoreType.DMA,
            pltpu.SemaphoreType.DMA,
            pltpu.SemaphoreType.DMA,
            pltpu.SemaphoreType.DMA,
        ],
        compiler_params=pltpu.CompilerParams(
            use_tc_tiling_on_sc=True, needs_layout_passes=False),
    )
    def emb_kernel(x_hbm, table_hbm, out_hbm, idx_v, idxq_v,
                   rows0, rows1, blk0, blk1, sg0, sg1, so0, so1):
        wid = lax.axis_index("s") * NC + lax.axis_index("c")
        g0 = wid * n_blocks
        pltpu.sync_copy(x_hbm.at[pl.ds(g0 * BLK, b_per_w)], idx_v)

        def quo(k, carry):
            v = idx_v[pl.ds(k * L, L)]
            idxq_v[pl.ds(k * L, L)] = lax.shift_right_logical(v, 2)
            return carry

        lax.fori_loop(0, b_per_w // L, quo, 0)

        iota = lax.iota(jnp.int32, L)

        def gather(i, rows, sem):
            return pltpu.make_async_copy(
                table_hbm.at[idxq_v.at[pl.ds(i * BLK, BLK)]], rows, sem)

        def store(i, blk, sem):
            g = g0 + i
            s = g // (16384 // BLK)
            b0 = (g % (16384 // BLK)) * BLK
            return pltpu.make_async_copy(
                blk, out_hbm.at[s, :, pl.ds(b0, BLK)], sem)

        def extract(i, rows, blk):
            cols = []
            rowvs = []
            for g8 in range(n_groups):
                v = idx_v[pl.ds(i * BLK + g8 * L, L)]
                cols.append((v & (PACK - 1)) * D)
                rowvs.append(iota + g8 * L)
            for d in range(D):
                for g8 in range(n_groups):
                    vals = plsc.load_gather(rows, [rowvs[g8], cols[g8] + d])
                    blk[d, pl.ds(g8 * L, L)] = vals

        gather(0, rows0, sg0).start()

        def step(k, carry):
            for p in range(2):
                i = 2 * k + p
                rows, blk, sgp, sop = (
                    (rows0, blk0, sg0, so0) if p == 0
                    else (rows1, blk1, sg1, so1))
                nrows, sgn, son = (
                    (rows1, sg1, so1) if p == 0 else (rows0, sg0, so0))
                nblk = blk1 if p == 0 else blk0

                @pl.when(i + 1 < n_blocks)
                def _():
                    @pl.when(i >= 1)
                    def _():
                        store(i - 1, nblk, son).wait()

                    gather(i + 1, nrows, sgn).start()

                gather(i, rows, sgp).wait()
                extract(i, rows, blk)
                store(i, blk, sop).start()

            return carry

        lax.fori_loop(0, n_blocks // 2, step, 0)
        store(n_blocks - 2, blk0, so0).wait()
        store(n_blocks - 1, blk1, so1).wait()

    return emb_kernel


def kernel(x, table):
    B0, S = x.shape
    V, D = table.shape
    B = B0 * S
    xf = jnp.transpose(x).reshape(-1).astype(jnp.int32)
    table4 = table.reshape(V * D // 128, 128)
    out_t = _build(B, D, V, table.dtype)(xf, table4)
    return jnp.transpose(out_t, (2, 0, 1))
